# Initial kernel scaffold; baseline (speedup 1.0000x reference)
import jax
import jax.numpy as jnp
from jax.experimental import pallas as pl


def kernel(X, graph_conv_filters_input, W1, b1, g1, be1, W2, b2, g2, be2, W3, b3, g3, be3):
    raise NotImplementedError("write your pallas kernel here")


# speedup vs baseline: 1.5942x; 1.5942x over previous
import functools

import jax
import jax.numpy as jnp
from jax.experimental import pallas as pl
from jax.experimental.pallas import tpu as pltpu

NUM_FILTERS = 2
N = 2048
D = 128
EPS = 1e-5


def _layer_norm(x, g, b):
    m = jnp.mean(x, axis=-1, keepdims=True)
    v = jnp.mean((x - m) ** 2, axis=-1, keepdims=True)
    return (x - m) / jnp.sqrt(v + EPS) * g + b


def _body(x_ref, f_ref, w1_ref, b1_ref, g1_ref, be1_ref,
          w2_ref, b2_ref, g2_ref, be2_ref,
          w3_ref, b3_ref, g3_ref, be3_ref, o_ref):
    x = x_ref[...]
    f = f_ref[...]

    def conv_layer(h, w_ref, b_ref):
        c = jnp.dot(f, h, preferred_element_type=jnp.float32)
        w = w_ref[...]
        z = (jnp.dot(c[:N], w[:D], preferred_element_type=jnp.float32)
             + jnp.dot(c[N:], w[D:], preferred_element_type=jnp.float32)
             + b_ref[...])
        return jax.nn.relu(z)

    h = conv_layer(x, w1_ref, b1_ref)
    h = _layer_norm(h, g1_ref[...], be1_ref[...])
    h = conv_layer(h, w2_ref, b2_ref)
    h = _layer_norm(h, g2_ref[...], be2_ref[...])
    h = conv_layer(h, w3_ref, b3_ref)
    out = _layer_norm(x + h, g3_ref[...], be3_ref[...])
    o_ref[...] = jax.nn.relu(out)


@functools.partial(jax.jit)
def _run(X, filt, W1, b1, g1, be1, W2, b2, g2, be2, W3, b3, g3, be3):
    x2 = X.reshape(N, D)
    f2 = filt.reshape(NUM_FILTERS * N, N)
    vecs = [v.reshape(1, D) for v in (b1, g1, be1, b2, g2, be2, b3, g3, be3)]
    b1r, g1r, be1r, b2r, g2r, be2r, b3r, g3r, be3r = vecs
    out = pl.pallas_call(
        _body,
        out_shape=jax.ShapeDtypeStruct((N, D), jnp.float32),
        compiler_params=pltpu.CompilerParams(
            vmem_limit_bytes=100 * 1024 * 1024,
        ),
    )(x2, f2, W1, b1r, g1r, be1r, W2, b2r, g2r, be2r, W3, b3r, g3r, be3r)
    return out.reshape(1, N, D)


def kernel(X, graph_conv_filters_input, W1, b1, g1, be1,
           W2, b2, g2, be2, W3, b3, g3, be3):
    return _run(X, graph_conv_filters_input, W1, b1, g1, be1,
                W2, b2, g2, be2, W3, b3, g3, be3)
